# baseline (speedup 1.0000x reference)
import jax
import jax.numpy as jnp
from jax.experimental import pallas as pl
from jax.experimental.pallas import tpu as pltpu

D0, D1, D2 = 1024, 1024, 128


_K = 16
_RPC = D0 // _K


def _gather_body(idx_ref, in_ref, out_ref, sem):
    idx = idx_ref[0]
    copies = [
        pltpu.make_async_copy(
            in_ref.at[pl.ds(k * _RPC, _RPC), idx],
            out_ref.at[pl.ds(k * _RPC, _RPC)],
            sem.at[k],
        )
        for k in range(_K)
    ]
    for c in copies:
        c.start()
    for c in copies:
        c.wait()


def kernel(input, indices):
    idx = indices.astype(jnp.int32).reshape((1,))
    return pl.pallas_call(
        _gather_body,
        in_specs=[
            pl.BlockSpec(memory_space=pltpu.SMEM),
            pl.BlockSpec(memory_space=pl.ANY),
        ],
        out_specs=pl.BlockSpec(memory_space=pl.ANY),
        out_shape=jax.ShapeDtypeStruct((D0, D2), jnp.float32),
        scratch_shapes=[pltpu.SemaphoreType.DMA((_K,))],
    )(idx, input)


# speedup vs baseline: 7.1800x; 7.1800x over previous
import jax
import jax.numpy as jnp
from jax.experimental import pallas as pl
from jax.experimental.pallas import tpu as pltpu

D0, D1, D2 = 1024, 1024, 128


def _gather_body(idx_ref, in_ref, out_ref, sem):
    idx = idx_ref[0]
    copy = pltpu.make_async_copy(in_ref.at[:, idx], out_ref, sem)
    copy.start()
    copy.wait()


def kernel(input, indices):
    idx = indices.astype(jnp.int32).reshape((1,))
    return pl.pallas_call(
        _gather_body,
        in_specs=[
            pl.BlockSpec(memory_space=pltpu.SMEM),
            pl.BlockSpec(memory_space=pl.ANY),
        ],
        out_specs=pl.BlockSpec(memory_space=pltpu.VMEM),
        out_shape=jax.ShapeDtypeStruct((D0, D2), jnp.float32),
        scratch_shapes=[pltpu.SemaphoreType.DMA],
    )(idx, input)
